# Initial kernel scaffold; baseline (speedup 1.0000x reference)
import jax
import jax.numpy as jnp
from jax.experimental import pallas as pl


def kernel(loc_ids, user_ids, loc_table, user_table, W, b, gamma, beta):
    raise NotImplementedError("write your pallas kernel here")


# speedup vs baseline: 1.6127x; 1.6127x over previous
import functools

import jax
import jax.numpy as jnp
from jax.experimental import pallas as pl
from jax.experimental.pallas import tpu as pltpu
from jax.experimental.pallas import tpu_sc as plsc

_B, _L = 4096, 200
_LOC_DIM, _USER_DIM, _HID = 64, 16, 80
_N = _B * _L

_GW = 512
_ROWS = 2048


def _sc_gather(loc_table, user_table, lids, uids):
    mesh = plsc.VectorSubcoreMesh(core_axis_name="core", subcore_axis_name="subcore")

    @functools.partial(
        pl.kernel,
        out_type=(
            jax.ShapeDtypeStruct((_N, _LOC_DIM), jnp.float32),
            jax.ShapeDtypeStruct((_N, _USER_DIM), jnp.float32),
        ),
        mesh=mesh,
        compiler_params=pltpu.CompilerParams(use_tc_tiling_on_sc=False),
    )
    def gather_kernel(loc_hbm, user_hbm, lid_hbm, uid_hbm, oloc_hbm, ouser_hbm):
        def body(li_vmem, ui_vmem, ol_vmem, ou_vmem):
            pltpu.sync_copy(loc_hbm.at[li_vmem.at[0]], ol_vmem)
            pltpu.sync_copy(user_hbm.at[ui_vmem.at[0]], ou_vmem)

        pltpu.emit_pipeline(
            body,
            grid=(_N // _GW,),
            in_specs=[
                pl.BlockSpec((1, _GW), lambda i: (0, i)),
                pl.BlockSpec((1, _GW), lambda i: (0, i)),
            ],
            out_specs=[
                pl.BlockSpec((_GW, _LOC_DIM), lambda i: (i, 0)),
                pl.BlockSpec((_GW, _USER_DIM), lambda i: (i, 0)),
            ],
            core_axis_name=("core", "subcore"),
            dimension_semantics=(pltpu.PARALLEL,),
        )(lid_hbm, uid_hbm, oloc_hbm, ouser_hbm)

    return gather_kernel(loc_table, user_table, lids, uids)


def _mlp_body(loc_ref, user_ref, wl_ref, wu_ref, b_ref, g_ref, bt_ref, out_ref):
    h = jnp.dot(loc_ref[...], wl_ref[...],
                preferred_element_type=jnp.float32,
                precision=jax.lax.Precision.HIGHEST)
    h += jnp.dot(user_ref[...], wu_ref[...],
                 preferred_element_type=jnp.float32,
                 precision=jax.lax.Precision.HIGHEST)
    h += b_ref[...]
    mu = jnp.mean(h, axis=1, keepdims=True)
    var = jnp.mean((h - mu) ** 2, axis=1, keepdims=True)
    y = (h - mu) * jax.lax.rsqrt(var + 1e-5) * g_ref[...] + bt_ref[...]
    out_ref[...] = 0.5 * y * (1.0 + jax.lax.erf(y * 0.7071067811865476))


def _mlp(loc_emb, user_emb, wl, wu, b, gamma, beta):
    return pl.pallas_call(
        _mlp_body,
        grid=(_N // _ROWS,),
        in_specs=[
            pl.BlockSpec((_ROWS, _LOC_DIM), lambda i: (i, 0)),
            pl.BlockSpec((_ROWS, _USER_DIM), lambda i: (i, 0)),
            pl.BlockSpec((_LOC_DIM, _HID), lambda i: (0, 0)),
            pl.BlockSpec((_USER_DIM, _HID), lambda i: (0, 0)),
            pl.BlockSpec((1, _HID), lambda i: (0, 0)),
            pl.BlockSpec((1, _HID), lambda i: (0, 0)),
            pl.BlockSpec((1, _HID), lambda i: (0, 0)),
        ],
        out_specs=pl.BlockSpec((_ROWS, _HID), lambda i: (i, 0)),
        out_shape=jax.ShapeDtypeStruct((_N, _HID), jnp.float32),
        compiler_params=pltpu.CompilerParams(
            dimension_semantics=("arbitrary",)),
    )(loc_emb, user_emb, wl, wu, b, gamma, beta)


def kernel(loc_ids, user_ids, loc_table, user_table, W, b, gamma, beta):
    lids = loc_ids.reshape(1, _N)
    uids = user_ids.reshape(1, _N)
    loc_emb, user_emb = _sc_gather(loc_table, user_table, lids, uids)
    wl = W[:, :_LOC_DIM].T
    wu = W[:, _LOC_DIM:].T
    out = _mlp(loc_emb, user_emb, wl, wu,
               b.reshape(1, _HID), gamma.reshape(1, _HID),
               beta.reshape(1, _HID))
    return out.reshape(_B, _L, _HID)


# speedup vs baseline: 1.8166x; 1.1264x over previous
import functools

import jax
import jax.numpy as jnp
from jax.experimental import pallas as pl
from jax.experimental.pallas import tpu as pltpu
from jax.experimental.pallas import tpu_sc as plsc

_B, _L = 4096, 200
_LOC_DIM, _USER_DIM, _HID = 64, 16, 80
_N = _B * _L

_GW = 512
_ROWS = 2048


def _sc_gather(loc_table, user_table, lids, uids):
    mesh = plsc.VectorSubcoreMesh(core_axis_name="core", subcore_axis_name="subcore")

    @functools.partial(
        pl.kernel,
        out_type=(
            jax.ShapeDtypeStruct((_N, _LOC_DIM), jnp.float32),
            jax.ShapeDtypeStruct((_N, _USER_DIM), jnp.float32),
        ),
        mesh=mesh,
        compiler_params=pltpu.CompilerParams(use_tc_tiling_on_sc=False),
    )
    def gather_kernel(loc_hbm, user_hbm, lid_hbm, uid_hbm, oloc_hbm, ouser_hbm):
        def body(li_vmem, ui_vmem, ol_vmem, ou_vmem):
            pltpu.sync_copy(loc_hbm.at[li_vmem.at[0]], ol_vmem)
            pltpu.sync_copy(user_hbm.at[ui_vmem.at[0]], ou_vmem)

        pltpu.emit_pipeline(
            body,
            grid=(_N // _GW,),
            in_specs=[
                pl.BlockSpec((1, _GW), lambda i: (0, i)),
                pl.BlockSpec((1, _GW), lambda i: (0, i)),
            ],
            out_specs=[
                pl.BlockSpec((_GW, _LOC_DIM), lambda i: (i, 0)),
                pl.BlockSpec((_GW, _USER_DIM), lambda i: (i, 0)),
            ],
            core_axis_name=("core", "subcore"),
            dimension_semantics=(pltpu.PARALLEL,),
        )(lid_hbm, uid_hbm, oloc_hbm, ouser_hbm)

    return gather_kernel(loc_table, user_table, lids, uids)


def _mlp_body(loc_ref, user_ref, wl_ref, wu_ref, b_ref, g_ref, bt_ref, out_ref):
    h = jnp.dot(loc_ref[...], wl_ref[...],
                preferred_element_type=jnp.float32)
    h += jnp.dot(user_ref[...], wu_ref[...],
                 preferred_element_type=jnp.float32)
    h += b_ref[...]
    mu = jnp.mean(h, axis=1, keepdims=True)
    var = jnp.mean((h - mu) ** 2, axis=1, keepdims=True)
    y = (h - mu) * jax.lax.rsqrt(var + 1e-5) * g_ref[...] + bt_ref[...]
    out_ref[...] = 0.5 * y * (1.0 + jax.lax.erf(y * 0.7071067811865476))


def _mlp(loc_emb, user_emb, wl, wu, b, gamma, beta):
    return pl.pallas_call(
        _mlp_body,
        grid=(_N // _ROWS,),
        in_specs=[
            pl.BlockSpec((_ROWS, _LOC_DIM), lambda i: (i, 0)),
            pl.BlockSpec((_ROWS, _USER_DIM), lambda i: (i, 0)),
            pl.BlockSpec((_LOC_DIM, _HID), lambda i: (0, 0)),
            pl.BlockSpec((_USER_DIM, _HID), lambda i: (0, 0)),
            pl.BlockSpec((1, _HID), lambda i: (0, 0)),
            pl.BlockSpec((1, _HID), lambda i: (0, 0)),
            pl.BlockSpec((1, _HID), lambda i: (0, 0)),
        ],
        out_specs=pl.BlockSpec((_ROWS, _HID), lambda i: (i, 0)),
        out_shape=jax.ShapeDtypeStruct((_N, _HID), jnp.float32),
        compiler_params=pltpu.CompilerParams(
            dimension_semantics=("arbitrary",)),
    )(loc_emb, user_emb, wl, wu, b, gamma, beta)


def kernel(loc_ids, user_ids, loc_table, user_table, W, b, gamma, beta):
    lids = loc_ids.reshape(1, _N)
    uids = user_ids.reshape(1, _N)
    loc_emb, user_emb = _sc_gather(loc_table, user_table, lids, uids)
    wl = W[:, :_LOC_DIM].T
    wu = W[:, _LOC_DIM:].T
    out = _mlp(loc_emb, user_emb, wl, wu,
               b.reshape(1, _HID), gamma.reshape(1, _HID),
               beta.reshape(1, _HID))
    return out.reshape(_B, _L, _HID)


# speedup vs baseline: 2.3741x; 1.3069x over previous
import functools

import jax
import jax.numpy as jnp
from jax import lax
from jax.experimental import pallas as pl
from jax.experimental.pallas import tpu as pltpu
from jax.experimental.pallas import tpu_sc as plsc

_B, _L = 4096, 200
_LOC_DIM, _USER_DIM, _HID = 64, 16, 80
_N = _B * _L

_NW = 32
_PER_W = _N // _NW
_C = 512
_NCHUNK = _PER_W // _C
_ROWS = 2048


def _sc_gather(loc_table, user_table, lids, uids):
    mesh = plsc.VectorSubcoreMesh(core_axis_name="core", subcore_axis_name="subcore")

    @functools.partial(
        pl.kernel,
        out_type=jax.ShapeDtypeStruct((_N, 128), jnp.float32),
        mesh=mesh,
        scratch_types=[
            pltpu.VMEM((_C,), jnp.int32),
            pltpu.VMEM((_C,), jnp.int32),
            pltpu.VMEM((_C, _LOC_DIM), jnp.float32),
            pltpu.VMEM((_C, _USER_DIM), jnp.float32),
            pltpu.SemaphoreType.DMA,
            pltpu.SemaphoreType.DMA,
        ],
        compiler_params=pltpu.CompilerParams(use_tc_tiling_on_sc=False),
    )
    def gather_kernel(loc_hbm, user_hbm, lid_hbm, uid_hbm, out_hbm,
                      li_v, ui_v, lrows_v, urows_v, sem_l, sem_u):
        wid = lax.axis_index("subcore") * 2 + lax.axis_index("core")
        base = wid * _PER_W

        @pl.loop(0, _NCHUNK)
        def _(c):
            row = base + c * _C
            pltpu.sync_copy(lid_hbm.at[pl.ds(row, _C)], li_v)
            pltpu.sync_copy(uid_hbm.at[pl.ds(row, _C)], ui_v)
            cl = pltpu.async_copy(loc_hbm.at[li_v], lrows_v, sem_l)
            cu = pltpu.async_copy(user_hbm.at[ui_v], urows_v, sem_u)
            cl.wait()
            cu.wait()
            pltpu.sync_copy(lrows_v, out_hbm.at[pl.ds(row, _C), pl.ds(0, _LOC_DIM)])
            pltpu.sync_copy(urows_v,
                            out_hbm.at[pl.ds(row, _C), pl.ds(_LOC_DIM, _USER_DIM)])

    return gather_kernel(loc_table, user_table, lids, uids)


def _mlp_body(emb_ref, wl_ref, wu_ref, b_ref, g_ref, bt_ref, out_ref):
    h = jnp.dot(emb_ref[:, :_LOC_DIM], wl_ref[...],
                preferred_element_type=jnp.float32)
    h += jnp.dot(emb_ref[:, _LOC_DIM:_LOC_DIM + _USER_DIM], wu_ref[...],
                 preferred_element_type=jnp.float32)
    h += b_ref[...]
    mu = jnp.mean(h, axis=1, keepdims=True)
    var = jnp.mean((h - mu) ** 2, axis=1, keepdims=True)
    y = (h - mu) * jax.lax.rsqrt(var + 1e-5) * g_ref[...] + bt_ref[...]
    out_ref[...] = 0.5 * y * (1.0 + jax.lax.erf(y * 0.7071067811865476))


def _mlp(emb, wl, wu, b, gamma, beta):
    return pl.pallas_call(
        _mlp_body,
        grid=(_N // _ROWS,),
        in_specs=[
            pl.BlockSpec((_ROWS, 128), lambda i: (i, 0)),
            pl.BlockSpec((_LOC_DIM, _HID), lambda i: (0, 0)),
            pl.BlockSpec((_USER_DIM, _HID), lambda i: (0, 0)),
            pl.BlockSpec((1, _HID), lambda i: (0, 0)),
            pl.BlockSpec((1, _HID), lambda i: (0, 0)),
            pl.BlockSpec((1, _HID), lambda i: (0, 0)),
        ],
        out_specs=pl.BlockSpec((_ROWS, _HID), lambda i: (i, 0)),
        out_shape=jax.ShapeDtypeStruct((_N, _HID), jnp.float32),
        compiler_params=pltpu.CompilerParams(
            dimension_semantics=("arbitrary",)),
    )(emb, wl, wu, b, gamma, beta)


def kernel(loc_ids, user_ids, loc_table, user_table, W, b, gamma, beta):
    lids = loc_ids.reshape(_N)
    uids = user_ids.reshape(_N)
    emb = _sc_gather(loc_table, user_table, lids, uids)
    wl = W[:, :_LOC_DIM].T
    wu = W[:, _LOC_DIM:].T
    out = _mlp(emb, wl, wu,
               b.reshape(1, _HID), gamma.reshape(1, _HID),
               beta.reshape(1, _HID))
    return out.reshape(_B, _L, _HID)


# speedup vs baseline: 2.3747x; 1.0003x over previous
import functools

import jax
import jax.numpy as jnp
from jax import lax
from jax.experimental import pallas as pl
from jax.experimental.pallas import tpu as pltpu
from jax.experimental.pallas import tpu_sc as plsc

_B, _L = 4096, 200
_LOC_DIM, _USER_DIM, _HID = 64, 16, 80
_N = _B * _L

_NW = 32
_PER_W = _N // _NW
_C = 512
_NCHUNK = _PER_W // _C
_ROWS = 3200


def _sc_gather(loc_table, user_table, lids, uids):
    mesh = plsc.VectorSubcoreMesh(core_axis_name="core", subcore_axis_name="subcore")

    @functools.partial(
        pl.kernel,
        out_type=jax.ShapeDtypeStruct((_N, 128), jnp.float32),
        mesh=mesh,
        scratch_types=[
            pltpu.VMEM((_C,), jnp.int32),
            pltpu.VMEM((_C,), jnp.int32),
            pltpu.VMEM((_C, _LOC_DIM), jnp.float32),
            pltpu.VMEM((_C, _USER_DIM), jnp.float32),
            pltpu.SemaphoreType.DMA,
            pltpu.SemaphoreType.DMA,
        ],
        compiler_params=pltpu.CompilerParams(use_tc_tiling_on_sc=False),
    )
    def gather_kernel(loc_hbm, user_hbm, lid_hbm, uid_hbm, out_hbm,
                      li_v, ui_v, lrows_v, urows_v, sem_l, sem_u):
        wid = lax.axis_index("subcore") * 2 + lax.axis_index("core")
        base = wid * _PER_W

        @pl.loop(0, _NCHUNK)
        def _(c):
            row = base + c * _C
            pltpu.sync_copy(lid_hbm.at[pl.ds(row, _C)], li_v)
            pltpu.sync_copy(uid_hbm.at[pl.ds(row, _C)], ui_v)
            cl = pltpu.async_copy(loc_hbm.at[li_v], lrows_v, sem_l)
            cu = pltpu.async_copy(user_hbm.at[ui_v], urows_v, sem_u)
            cl.wait()
            cu.wait()
            pltpu.sync_copy(lrows_v, out_hbm.at[pl.ds(row, _C), pl.ds(0, _LOC_DIM)])
            pltpu.sync_copy(urows_v,
                            out_hbm.at[pl.ds(row, _C), pl.ds(_LOC_DIM, _USER_DIM)])

    return gather_kernel(loc_table, user_table, lids, uids)


def _mlp_body(emb_ref, wl_ref, wu_ref, b_ref, g_ref, bt_ref, out_ref):
    h = jnp.dot(emb_ref[:, :_LOC_DIM], wl_ref[...],
                preferred_element_type=jnp.float32)
    h += jnp.dot(emb_ref[:, _LOC_DIM:_LOC_DIM + _USER_DIM], wu_ref[...],
                 preferred_element_type=jnp.float32)
    h += b_ref[...]
    mu = jnp.mean(h, axis=1, keepdims=True)
    var = jnp.mean((h - mu) ** 2, axis=1, keepdims=True)
    y = (h - mu) * jax.lax.rsqrt(var + 1e-5) * g_ref[...] + bt_ref[...]
    g = 0.5 * y * (1.0 + jax.lax.erf(y * 0.7071067811865476))
    out_ref[...] = g.reshape(out_ref.shape)


def _mlp(emb, wl, wu, b, gamma, beta):
    bt = _ROWS // _L
    return pl.pallas_call(
        _mlp_body,
        grid=(_B // bt,),
        in_specs=[
            pl.BlockSpec((_ROWS, 128), lambda i: (i, 0)),
            pl.BlockSpec((_LOC_DIM, _HID), lambda i: (0, 0)),
            pl.BlockSpec((_USER_DIM, _HID), lambda i: (0, 0)),
            pl.BlockSpec((1, _HID), lambda i: (0, 0)),
            pl.BlockSpec((1, _HID), lambda i: (0, 0)),
            pl.BlockSpec((1, _HID), lambda i: (0, 0)),
        ],
        out_specs=pl.BlockSpec((bt, _L, _HID), lambda i: (i, 0, 0)),
        out_shape=jax.ShapeDtypeStruct((_B, _L, _HID), jnp.float32),
        compiler_params=pltpu.CompilerParams(
            dimension_semantics=("arbitrary",)),
    )(emb, wl, wu, b, gamma, beta)


def kernel(loc_ids, user_ids, loc_table, user_table, W, b, gamma, beta):
    lids = loc_ids.reshape(_N)
    uids = user_ids.reshape(_N)
    emb = _sc_gather(loc_table, user_table, lids, uids)
    wl = W[:, :_LOC_DIM].T
    wu = W[:, _LOC_DIM:].T
    return _mlp(emb, wl, wu,
                b.reshape(1, _HID), gamma.reshape(1, _HID),
                beta.reshape(1, _HID))


# speedup vs baseline: 3.5372x; 1.4895x over previous
import functools

import jax
import jax.numpy as jnp
from jax import lax
from jax.experimental import pallas as pl
from jax.experimental.pallas import tpu as pltpu
from jax.experimental.pallas import tpu_sc as plsc

_B, _L = 4096, 200
_LOC_DIM, _USER_DIM, _HID = 64, 16, 80
_N = _B * _L

_NW = 32
_PER_W = _N // _NW
_C = 512
_NCHUNK = _PER_W // _C
_LCH = 4


def _sc_gather(loc_table, user_table, lids, uids):
    mesh = plsc.VectorSubcoreMesh(core_axis_name="core", subcore_axis_name="subcore")

    @functools.partial(
        pl.kernel,
        out_type=jax.ShapeDtypeStruct((_N, 128), jnp.float32),
        mesh=mesh,
        scratch_types=[
            pltpu.VMEM((_C,), jnp.int32),
            pltpu.VMEM((_C,), jnp.int32),
            pltpu.VMEM((_C, _LOC_DIM), jnp.float32),
            pltpu.VMEM((_C, _USER_DIM), jnp.float32),
            pltpu.SemaphoreType.DMA,
            pltpu.SemaphoreType.DMA,
        ],
        compiler_params=pltpu.CompilerParams(use_tc_tiling_on_sc=False),
    )
    def gather_kernel(loc_hbm, user_hbm, lid_hbm, uid_hbm, out_hbm,
                      li_v, ui_v, lrows_v, urows_v, sem_l, sem_u):
        wid = lax.axis_index("subcore") * 2 + lax.axis_index("core")
        base = wid * _PER_W

        @pl.loop(0, _NCHUNK)
        def _(c):
            row = base + c * _C
            pltpu.sync_copy(lid_hbm.at[pl.ds(row, _C)], li_v)
            pltpu.sync_copy(uid_hbm.at[pl.ds(row, _C)], ui_v)
            cl = pltpu.async_copy(loc_hbm.at[li_v], lrows_v, sem_l)
            cu = pltpu.async_copy(user_hbm.at[ui_v], urows_v, sem_u)
            cl.wait()
            cu.wait()
            pltpu.sync_copy(lrows_v, out_hbm.at[pl.ds(row, _C), pl.ds(0, _LOC_DIM)])
            pltpu.sync_copy(urows_v,
                            out_hbm.at[pl.ds(row, _C), pl.ds(_LOC_DIM, _USER_DIM)])

    return gather_kernel(loc_table, user_table, lids, uids)


def _mlp_body(emb_ref, wl_ref, wu_ref, b_ref, g_ref, bt_ref, out_ref):
    for l in range(_LCH):
        e = emb_ref[pl.ds(l * _B, _B), :]
        hT = jax.lax.dot_general(
            wl_ref[...], e[:, :_LOC_DIM],
            dimension_numbers=(((1,), (1,)), ((), ())),
            preferred_element_type=jnp.float32)
        hT += jax.lax.dot_general(
            wu_ref[...], e[:, _LOC_DIM:_LOC_DIM + _USER_DIM],
            dimension_numbers=(((1,), (1,)), ((), ())),
            preferred_element_type=jnp.float32)
        hT += b_ref[...]
        mu = jnp.mean(hT, axis=0, keepdims=True)
        var = jnp.mean((hT - mu) ** 2, axis=0, keepdims=True)
        y = (hT - mu) * jax.lax.rsqrt(var + 1e-5) * g_ref[...] + bt_ref[...]
        out_ref[l] = 0.5 * y * (1.0 + jax.lax.erf(y * 0.7071067811865476))


def _mlp(emb, wl, wu, b, gamma, beta):
    out3 = pl.pallas_call(
        _mlp_body,
        grid=(_L // _LCH,),
        in_specs=[
            pl.BlockSpec((_LCH * _B, 128), lambda i: (i, 0)),
            pl.BlockSpec((_HID, _LOC_DIM), lambda i: (0, 0)),
            pl.BlockSpec((_HID, _USER_DIM), lambda i: (0, 0)),
            pl.BlockSpec((_HID, 1), lambda i: (0, 0)),
            pl.BlockSpec((_HID, 1), lambda i: (0, 0)),
            pl.BlockSpec((_HID, 1), lambda i: (0, 0)),
        ],
        out_specs=pl.BlockSpec((_LCH, _HID, _B), lambda i: (i, 0, 0)),
        out_shape=jax.ShapeDtypeStruct((_L, _HID, _B), jnp.float32),
        compiler_params=pltpu.CompilerParams(
            dimension_semantics=("arbitrary",)),
    )(emb, wl, wu, b, gamma, beta)
    return out3.transpose(2, 0, 1)


def kernel(loc_ids, user_ids, loc_table, user_table, W, b, gamma, beta):
    lids = loc_ids.T.reshape(_N)
    uids = user_ids.T.reshape(_N)
    emb = _sc_gather(loc_table, user_table, lids, uids)
    wl = W[:, :_LOC_DIM]
    wu = W[:, _LOC_DIM:]
    return _mlp(emb, wl, wu,
                b.reshape(_HID, 1), gamma.reshape(_HID, 1),
                beta.reshape(_HID, 1))
